# baseline (speedup 1.0000x reference)
import functools

import jax
import jax.numpy as jnp
from jax import lax
from jax.experimental import pallas as pl
from jax.experimental.pallas import tpu as pltpu

D_MODEL = 768
N_HEAD = 12
D_HEAD = D_MODEL // N_HEAD
SEQ = 2048
_SCALE = 1.0 / (D_HEAD ** 0.5)
_CH = 512
_NCH = SEQ // _CH


def _dot_t(a, b):
    return lax.dot_general(a, b, (((1,), (1,)), ((), ())),
                           preferred_element_type=jnp.float32)


def _dot(a, b):
    return lax.dot_general(a, b, (((1,), (0,)), ((), ())),
                           preferred_element_type=jnp.float32)


def _proj_kernel(kv_ref, wkv_ref, k_ref, v1_ref, kv1_ref):
    kvp = _dot_t(kv_ref[...], wkv_ref[...])
    kk = kvp[:, :D_HEAD]
    vv = kvp[:, D_HEAD:]
    kn = jnp.sqrt(jnp.sum(kk * kk, axis=0, keepdims=True))
    vn = jnp.sqrt(jnp.sum(vv * vv, axis=0, keepdims=True))
    kk = kk / jnp.maximum(kn, 1e-12)
    vv = vv / jnp.maximum(vn, 1e-12)
    ones = jnp.ones((SEQ, 1), jnp.float32)
    k_ref[...] = kk
    v1_ref[...] = jnp.concatenate([vv, ones], axis=1)
    kv1_ref[...] = jnp.concatenate([kk, vv, ones], axis=1)


def _head(qh, k, v1, kv1, gate):
    s = _dot_t(qh, k)
    m = jnp.max(s, axis=1, keepdims=True)
    rkv = jnp.zeros((SEQ, 2 * D_HEAD + 1), jnp.float32)
    pv = jnp.zeros((SEQ, D_HEAD + 1), jnp.float32)
    for c in range(_NCH):
        sc = s[:, c * _CH:(c + 1) * _CH]
        ohc = (sc >= m).astype(jnp.float32)
        pc = jnp.exp(sc * _SCALE)
        rkv = rkv + _dot(ohc, kv1[c * _CH:(c + 1) * _CH, :])
        pv = pv + _dot(pc, v1[c * _CH:(c + 1) * _CH, :])
    local_out = pv[:, :D_HEAD] / pv[:, D_HEAD:]
    pr = jnp.zeros((SEQ, D_HEAD + 1), jnp.float32)
    for c in range(_NCH):
        rkvc = rkv[c * _CH:(c + 1) * _CH, :]
        s2c = _dot_t(qh, rkvc[:, :D_HEAD])
        p2c = jnp.exp(s2c * _SCALE)
        pr = pr + _dot(p2c, rkvc[:, D_HEAD:])
    r_out = pr[:, :D_HEAD] / pr[:, D_HEAD:]
    return r_out * gate + local_out * (1.0 - gate)


def _main_kernel(q_ref, k_ref, v1_ref, kv1_ref, wq_ref, wct_ref, bias_ref,
                 out_ref):
    t = pl.program_id(0)
    gate = jax.nn.sigmoid(bias_ref[...])
    k = k_ref[...]
    v1 = v1_ref[...]
    kv1 = kv1_ref[...]

    qh2 = _dot_t(q_ref[...], wq_ref[...])
    outs = [_head(qh2[:, i * D_HEAD:(i + 1) * D_HEAD], k, v1, kv1, gate)
            for i in range(3)]
    contrib = _dot(jnp.concatenate(outs, axis=1), wct_ref[...])

    @pl.when(t == 0)
    def _init():
        out_ref[...] = contrib

    @pl.when(t != 0)
    def _acc():
        out_ref[...] += contrib


@functools.partial(jax.jit, static_argnames=())
def kernel(q, kv, w_q, w_kv, w_concat, bias):
    b, l, dm = q.shape
    q2 = q.reshape(l, dm)
    kv2 = kv.reshape(l, dm)
    wct = w_concat.T
    bias2 = bias.reshape(1, D_HEAD)

    k_n, v1_n, kv1_n = pl.pallas_call(
        _proj_kernel,
        out_shape=[
            jax.ShapeDtypeStruct((l, D_HEAD), jnp.float32),
            jax.ShapeDtypeStruct((l, D_HEAD + 1), jnp.float32),
            jax.ShapeDtypeStruct((l, 2 * D_HEAD + 1), jnp.float32),
        ],
    )(kv2, w_kv)

    out = pl.pallas_call(
        _main_kernel,
        grid=(N_HEAD // 3,),
        in_specs=[
            pl.BlockSpec((l, dm), lambda t: (0, 0)),
            pl.BlockSpec((l, D_HEAD), lambda t: (0, 0)),
            pl.BlockSpec((l, D_HEAD + 1), lambda t: (0, 0)),
            pl.BlockSpec((l, 2 * D_HEAD + 1), lambda t: (0, 0)),
            pl.BlockSpec((3 * D_HEAD, dm), lambda t: (t, 0)),
            pl.BlockSpec((3 * D_HEAD, dm), lambda t: (t, 0)),
            pl.BlockSpec((1, D_HEAD), lambda t: (0, 0)),
        ],
        out_specs=pl.BlockSpec((l, dm), lambda t: (0, 0)),
        out_shape=jax.ShapeDtypeStruct((l, dm), jnp.float32),
        compiler_params=pltpu.CompilerParams(
            dimension_semantics=("arbitrary",),
        ),
    )(q2, k_n, v1_n, kv1_n, w_q, wct, bias2)
    return out.reshape(b, l, dm)


# speedup vs baseline: 1.2142x; 1.2142x over previous
import functools

import jax
import jax.numpy as jnp
from jax import lax
from jax.experimental import pallas as pl
from jax.experimental.pallas import tpu as pltpu

D_MODEL = 768
N_HEAD = 12
D_HEAD = D_MODEL // N_HEAD
SEQ = 2048
_SCALE = 1.0 / (D_HEAD ** 0.5)
_CH = 512
_NCH = SEQ // _CH


def _dot_t(a, b):
    return lax.dot_general(a, b, (((1,), (1,)), ((), ())),
                           preferred_element_type=jnp.float32)


def _dot(a, b):
    return lax.dot_general(a, b, (((1,), (0,)), ((), ())),
                           preferred_element_type=jnp.float32)


def _proj_kernel(kv_ref, wkv_ref, k_ref, v1_ref, kv1_ref):
    kvp = _dot_t(kv_ref[...], wkv_ref[...])
    kk = kvp[:, :D_HEAD]
    vv = kvp[:, D_HEAD:]
    kn = jnp.sqrt(jnp.sum(kk * kk, axis=0, keepdims=True))
    vn = jnp.sqrt(jnp.sum(vv * vv, axis=0, keepdims=True))
    kk = kk / jnp.maximum(kn, 1e-12)
    vv = vv / jnp.maximum(vn, 1e-12)
    ones = jnp.ones((SEQ, 1), jnp.float32)
    k_ref[...] = kk
    v1_ref[...] = jnp.concatenate([vv, ones], axis=1)
    kv1_ref[...] = jnp.concatenate([kk, vv, ones], axis=1)


def _head(qh, k, v1, kv1, gate):
    s = _dot_t(qh, k)
    m = jnp.max(s, axis=1, keepdims=True)
    rkv = jnp.zeros((SEQ, 2 * D_HEAD + 1), jnp.float32)
    pv = jnp.zeros((SEQ, D_HEAD + 1), jnp.float32)
    for c in range(_NCH):
        sc = s[:, c * _CH:(c + 1) * _CH]
        ohc = (sc >= m).astype(jnp.bfloat16)
        pc = jnp.exp(sc * _SCALE).astype(jnp.bfloat16)
        rkv = rkv + _dot(ohc, kv1[c * _CH:(c + 1) * _CH, :].astype(jnp.bfloat16))
        pv = pv + _dot(pc, v1[c * _CH:(c + 1) * _CH, :].astype(jnp.bfloat16))
    local_out = pv[:, :D_HEAD] / pv[:, D_HEAD:]
    pr = jnp.zeros((SEQ, D_HEAD + 1), jnp.float32)
    for c in range(_NCH):
        rkvc = rkv[c * _CH:(c + 1) * _CH, :]
        s2c = _dot_t(qh, rkvc[:, :D_HEAD])
        p2c = jnp.exp(s2c * _SCALE).astype(jnp.bfloat16)
        pr = pr + _dot(p2c, rkvc[:, D_HEAD:].astype(jnp.bfloat16))
    r_out = pr[:, :D_HEAD] / pr[:, D_HEAD:]
    return r_out * gate + local_out * (1.0 - gate)


def _main_kernel(q_ref, k_ref, v1_ref, kv1_ref, wq_ref, wct_ref, bias_ref,
                 out_ref):
    t = pl.program_id(0)
    gate = jax.nn.sigmoid(bias_ref[...])
    k = k_ref[...]
    v1 = v1_ref[...]
    kv1 = kv1_ref[...]

    qh2 = _dot_t(q_ref[...], wq_ref[...])
    out_a = _head(qh2[:, :D_HEAD], k, v1, kv1, gate)
    out_b = _head(qh2[:, D_HEAD:], k, v1, kv1, gate)
    contrib = _dot(jnp.concatenate([out_a, out_b], axis=1), wct_ref[...])

    @pl.when(t == 0)
    def _init():
        out_ref[...] = contrib

    @pl.when(t != 0)
    def _acc():
        out_ref[...] += contrib


@functools.partial(jax.jit, static_argnames=())
def kernel(q, kv, w_q, w_kv, w_concat, bias):
    b, l, dm = q.shape
    q2 = q.reshape(l, dm)
    kv2 = kv.reshape(l, dm)
    wct = w_concat.T
    bias2 = bias.reshape(1, D_HEAD)

    k_n, v1_n, kv1_n = pl.pallas_call(
        _proj_kernel,
        out_shape=[
            jax.ShapeDtypeStruct((l, D_HEAD), jnp.float32),
            jax.ShapeDtypeStruct((l, D_HEAD + 1), jnp.float32),
            jax.ShapeDtypeStruct((l, 2 * D_HEAD + 1), jnp.float32),
        ],
    )(kv2, w_kv)

    out = pl.pallas_call(
        _main_kernel,
        grid=(N_HEAD // 2,),
        in_specs=[
            pl.BlockSpec((l, dm), lambda t: (0, 0)),
            pl.BlockSpec((l, D_HEAD), lambda t: (0, 0)),
            pl.BlockSpec((l, D_HEAD + 1), lambda t: (0, 0)),
            pl.BlockSpec((l, 2 * D_HEAD + 1), lambda t: (0, 0)),
            pl.BlockSpec((2 * D_HEAD, dm), lambda t: (t, 0)),
            pl.BlockSpec((2 * D_HEAD, dm), lambda t: (t, 0)),
            pl.BlockSpec((1, D_HEAD), lambda t: (0, 0)),
        ],
        out_specs=pl.BlockSpec((l, dm), lambda t: (0, 0)),
        out_shape=jax.ShapeDtypeStruct((l, dm), jnp.float32),
        compiler_params=pltpu.CompilerParams(
            dimension_semantics=("arbitrary",),
        ),
    )(q2, k_n, v1_n, kv1_n, w_q, wct, bias2)
    return out.reshape(b, l, dm)


# speedup vs baseline: 1.2410x; 1.0221x over previous
import functools

import jax
import jax.numpy as jnp
from jax import lax
from jax.experimental import pallas as pl
from jax.experimental.pallas import tpu as pltpu

D_MODEL = 768
N_HEAD = 12
D_HEAD = D_MODEL // N_HEAD
SEQ = 2048
_SCALE = 1.0 / (D_HEAD ** 0.5)
_CH = 1024
_NCH = SEQ // _CH


def _dot_t(a, b):
    return lax.dot_general(a, b, (((1,), (1,)), ((), ())),
                           preferred_element_type=jnp.float32)


def _dot(a, b):
    return lax.dot_general(a, b, (((1,), (0,)), ((), ())),
                           preferred_element_type=jnp.float32)


def _proj_kernel(kv_ref, wkv_ref, k_ref, v1_ref, kv1_ref):
    kvp = _dot_t(kv_ref[...], wkv_ref[...])
    kk = kvp[:, :D_HEAD]
    vv = kvp[:, D_HEAD:]
    kn = jnp.sqrt(jnp.sum(kk * kk, axis=0, keepdims=True))
    vn = jnp.sqrt(jnp.sum(vv * vv, axis=0, keepdims=True))
    kk = kk / jnp.maximum(kn, 1e-12)
    vv = vv / jnp.maximum(vn, 1e-12)
    ones = jnp.ones((SEQ, 1), jnp.float32)
    k_ref[...] = kk
    v1_ref[...] = jnp.concatenate([vv, ones], axis=1)
    kv1_ref[...] = jnp.concatenate([kk, vv, ones], axis=1)


def _head(qh, k, v1, kv1, gate):
    s = _dot_t(qh, k)
    m = jnp.max(s, axis=1, keepdims=True)
    rkv = jnp.zeros((SEQ, 2 * D_HEAD + 1), jnp.float32)
    pv = jnp.zeros((SEQ, D_HEAD + 1), jnp.float32)
    for c in range(_NCH):
        sc = s[:, c * _CH:(c + 1) * _CH]
        ohc = (sc >= m).astype(jnp.float32)
        pc = jnp.exp(sc * _SCALE)
        rkv = rkv + _dot(ohc, kv1[c * _CH:(c + 1) * _CH, :])
        pv = pv + _dot(pc, v1[c * _CH:(c + 1) * _CH, :])
    local_out = pv[:, :D_HEAD] / pv[:, D_HEAD:]
    pr = jnp.zeros((SEQ, D_HEAD + 1), jnp.float32)
    for c in range(_NCH):
        rkvc = rkv[c * _CH:(c + 1) * _CH, :]
        s2c = _dot_t(qh, rkvc[:, :D_HEAD])
        p2c = jnp.exp(s2c * _SCALE)
        pr = pr + _dot(p2c, rkvc[:, D_HEAD:])
    r_out = pr[:, :D_HEAD] / pr[:, D_HEAD:]
    return r_out * gate + local_out * (1.0 - gate)


def _main_kernel(q_ref, k_ref, v1_ref, kv1_ref, wq_ref, wct_ref, bias_ref,
                 out_ref):
    t = pl.program_id(0)
    gate = jax.nn.sigmoid(bias_ref[...])
    k = k_ref[...]
    v1 = v1_ref[...]
    kv1 = kv1_ref[...]

    qh2 = _dot_t(q_ref[...], wq_ref[...])
    out_a = _head(qh2[:, :D_HEAD], k, v1, kv1, gate)
    out_b = _head(qh2[:, D_HEAD:], k, v1, kv1, gate)
    contrib = _dot(jnp.concatenate([out_a, out_b], axis=1), wct_ref[...])

    @pl.when(t == 0)
    def _init():
        out_ref[...] = contrib

    @pl.when(t != 0)
    def _acc():
        out_ref[...] += contrib


@functools.partial(jax.jit, static_argnames=())
def kernel(q, kv, w_q, w_kv, w_concat, bias):
    b, l, dm = q.shape
    q2 = q.reshape(l, dm)
    kv2 = kv.reshape(l, dm)
    wct = w_concat.T
    bias2 = bias.reshape(1, D_HEAD)

    k_n, v1_n, kv1_n = pl.pallas_call(
        _proj_kernel,
        out_shape=[
            jax.ShapeDtypeStruct((l, D_HEAD), jnp.float32),
            jax.ShapeDtypeStruct((l, D_HEAD + 1), jnp.float32),
            jax.ShapeDtypeStruct((l, 2 * D_HEAD + 1), jnp.float32),
        ],
    )(kv2, w_kv)

    out = pl.pallas_call(
        _main_kernel,
        grid=(N_HEAD // 2,),
        in_specs=[
            pl.BlockSpec((l, dm), lambda t: (0, 0)),
            pl.BlockSpec((l, D_HEAD), lambda t: (0, 0)),
            pl.BlockSpec((l, D_HEAD + 1), lambda t: (0, 0)),
            pl.BlockSpec((l, 2 * D_HEAD + 1), lambda t: (0, 0)),
            pl.BlockSpec((2 * D_HEAD, dm), lambda t: (t, 0)),
            pl.BlockSpec((2 * D_HEAD, dm), lambda t: (t, 0)),
            pl.BlockSpec((1, D_HEAD), lambda t: (0, 0)),
        ],
        out_specs=pl.BlockSpec((l, dm), lambda t: (0, 0)),
        out_shape=jax.ShapeDtypeStruct((l, dm), jnp.float32),
        compiler_params=pltpu.CompilerParams(
            dimension_semantics=("arbitrary",),
        ),
    )(q2, k_n, v1_n, kv1_n, w_q, wct, bias2)
    return out.reshape(b, l, dm)


# speedup vs baseline: 1.3363x; 1.0768x over previous
import functools

import jax
import jax.numpy as jnp
from jax import lax
from jax.experimental import pallas as pl
from jax.experimental.pallas import tpu as pltpu

D_MODEL = 768
N_HEAD = 12
D_HEAD = D_MODEL // N_HEAD
SEQ = 2048
_SCALE = 1.0 / (D_HEAD ** 0.5)
_CH = 256
_NCH = SEQ // _CH


def _dot_t(a, b):
    return lax.dot_general(a, b, (((1,), (1,)), ((), ())),
                           preferred_element_type=jnp.float32)


def _dot(a, b):
    return lax.dot_general(a, b, (((1,), (0,)), ((), ())),
                           preferred_element_type=jnp.float32)


def _proj_kernel(kv_ref, wkv_ref, k_ref, v1_ref, kv1_ref):
    kvp = _dot_t(kv_ref[...], wkv_ref[...])
    kk = kvp[:, :D_HEAD]
    vv = kvp[:, D_HEAD:]
    kn = jnp.sqrt(jnp.sum(kk * kk, axis=0, keepdims=True))
    vn = jnp.sqrt(jnp.sum(vv * vv, axis=0, keepdims=True))
    kk = kk / jnp.maximum(kn, 1e-12)
    vv = vv / jnp.maximum(vn, 1e-12)
    ones = jnp.ones((SEQ, 1), jnp.float32)
    k_ref[...] = kk
    v1_ref[...] = jnp.concatenate([vv, ones], axis=1)
    kv1_ref[...] = jnp.concatenate([kk, vv, ones], axis=1)


def _head(qh, k, v1, kv1, gate):
    s = _dot_t(qh, k)
    m = jnp.max(s, axis=1, keepdims=True)
    rkv = jnp.zeros((SEQ, 2 * D_HEAD + 1), jnp.float32)
    pv = jnp.zeros((SEQ, D_HEAD + 1), jnp.float32)
    for c in range(_NCH):
        sc = s[:, c * _CH:(c + 1) * _CH]
        ohc = (sc >= m).astype(jnp.float32)
        pc = jnp.exp(sc * _SCALE)
        rkv = rkv + _dot(ohc, kv1[c * _CH:(c + 1) * _CH, :])
        pv = pv + _dot(pc, v1[c * _CH:(c + 1) * _CH, :])
    local_out = pv[:, :D_HEAD] / pv[:, D_HEAD:]
    pr = jnp.zeros((SEQ, D_HEAD + 1), jnp.float32)
    for c in range(_NCH):
        rkvc = rkv[c * _CH:(c + 1) * _CH, :]
        s2c = _dot_t(qh, rkvc[:, :D_HEAD])
        p2c = jnp.exp(s2c * _SCALE)
        pr = pr + _dot(p2c, rkvc[:, D_HEAD:])
    r_out = pr[:, :D_HEAD] / pr[:, D_HEAD:]
    return r_out * gate + local_out * (1.0 - gate)


def _main_kernel(q_ref, k_ref, v1_ref, kv1_ref, wq_ref, wct_ref, bias_ref,
                 out_ref):
    t = pl.program_id(0)
    gate = jax.nn.sigmoid(bias_ref[...])
    k = k_ref[...]
    v1 = v1_ref[...]
    kv1 = kv1_ref[...]

    qh2 = _dot_t(q_ref[...], wq_ref[...])
    out_a = _head(qh2[:, :D_HEAD], k, v1, kv1, gate)
    out_b = _head(qh2[:, D_HEAD:], k, v1, kv1, gate)
    contrib = _dot(jnp.concatenate([out_a, out_b], axis=1), wct_ref[...])

    @pl.when(t == 0)
    def _init():
        out_ref[...] = contrib

    @pl.when(t != 0)
    def _acc():
        out_ref[...] += contrib


@functools.partial(jax.jit, static_argnames=())
def kernel(q, kv, w_q, w_kv, w_concat, bias):
    b, l, dm = q.shape
    q2 = q.reshape(l, dm)
    kv2 = kv.reshape(l, dm)
    wct = w_concat.T
    bias2 = bias.reshape(1, D_HEAD)

    k_n, v1_n, kv1_n = pl.pallas_call(
        _proj_kernel,
        out_shape=[
            jax.ShapeDtypeStruct((l, D_HEAD), jnp.float32),
            jax.ShapeDtypeStruct((l, D_HEAD + 1), jnp.float32),
            jax.ShapeDtypeStruct((l, 2 * D_HEAD + 1), jnp.float32),
        ],
    )(kv2, w_kv)

    out = pl.pallas_call(
        _main_kernel,
        grid=(N_HEAD // 2,),
        in_specs=[
            pl.BlockSpec((l, dm), lambda t: (0, 0)),
            pl.BlockSpec((l, D_HEAD), lambda t: (0, 0)),
            pl.BlockSpec((l, D_HEAD + 1), lambda t: (0, 0)),
            pl.BlockSpec((l, 2 * D_HEAD + 1), lambda t: (0, 0)),
            pl.BlockSpec((2 * D_HEAD, dm), lambda t: (t, 0)),
            pl.BlockSpec((2 * D_HEAD, dm), lambda t: (t, 0)),
            pl.BlockSpec((1, D_HEAD), lambda t: (0, 0)),
        ],
        out_specs=pl.BlockSpec((l, dm), lambda t: (0, 0)),
        out_shape=jax.ShapeDtypeStruct((l, dm), jnp.float32),
        compiler_params=pltpu.CompilerParams(
            dimension_semantics=("arbitrary",),
        ),
    )(q2, k_n, v1_n, kv1_n, w_q, wct, bias2)
    return out.reshape(b, l, dm)
